# baseline (speedup 1.0000x reference)
import jax
import jax.numpy as jnp
from jax import lax
from jax.experimental import pallas as pl

NUM_AGENTS = 32
ACTION_DIM = 8
NUM_BELIEF = 120
HIDDEN = 32
HEADS = 4
FEAT = HIDDEN * HEADS

BF = jnp.bfloat16

_W1, _W2, _VW, _OW, _LW, _AW = 0, 128, 256, 384, 512, 576
_B1, _B2, _VB, _OB, _LB, _APB = 832, 840, 848, 856, 864, 872
_ROWS = 888
_ATT_LANE = 72

_DN_T = (((1,), (1,)), ((), ()))


def _dot_t(x, w, out_dtype=jnp.float32):
    return lax.dot_general(x, w, _DN_T, preferred_element_type=out_dtype)


def _gat_block(h3, asf, adf, bias, nb):
    A, C, F = NUM_AGENTS, HIDDEN, FEAT
    hflat = h3.reshape(nb * A, F)
    bi = lax.broadcasted_iota(jnp.int32, (F, F), 0) // C
    bj = lax.broadcasted_iota(jnp.int32, (F, F), 1) // C
    BD = (bi == bj).astype(jnp.float32)
    BDb = BD.astype(BF)
    a_src_bc = jnp.dot(hflat, (asf * BD).astype(BF),
                       preferred_element_type=jnp.float32).astype(BF)
    a_dst_bc = jnp.dot(hflat, (adf * BD).astype(BF),
                       preferred_element_type=jnp.float32).astype(BF)
    sel = (lax.broadcasted_iota(jnp.int32, (A, F), 0)
           == lax.broadcasted_iota(jnp.int32, (A, F), 1) % C).astype(BF)
    asrcT = jnp.sum(a_src_bc.reshape(nb, A, F) * sel[None], axis=1)
    L = asrcT[:, None, :] + a_dst_bc.reshape(nb, A, F)
    L = jnp.maximum(L, jnp.asarray(0.2, BF) * L)
    j_io = lax.broadcasted_iota(jnp.int32, (nb, A, F), 1)
    i_io = lax.broadcasted_iota(jnp.int32, (nb, A, F), 2) % C
    E = jnp.where(j_io == i_io, jnp.asarray(0.0, BF), jnp.exp(L))
    den = jnp.dot(E.reshape(nb * A, F), BDb,
                  preferred_element_type=jnp.float32).reshape(nb, A, F)
    rden = jnp.ones((), BF) / den.astype(BF)
    Hbig = jnp.concatenate([h3, h3, h3, h3], axis=1)
    ri = lax.broadcasted_iota(jnp.int32, (nb, F, F), 1) // C
    ci = lax.broadcasted_iota(jnp.int32, (nb, F, F), 2) // C
    Hbig = jnp.where(ri == ci, Hbig, jnp.asarray(0.0, BF))
    outU = lax.dot_general(E, Hbig, (((2,), (1,)), ((0,), (0,))),
                           preferred_element_type=jnp.float32)
    out = outU.astype(BF) * rden + bias[None, :, :]
    return jnp.maximum(out, jnp.asarray(0.0, BF))


def _tgnn_kernel(sig_ref, na_ref, wp_ref, out_ref):
    nb = sig_ref.shape[0]
    A, F = NUM_AGENTS, FEAT
    w1 = wp_ref[_W1:_W1 + F, :]
    acts = na_ref[...].reshape(nb * A, ACTION_DIM).astype(BF)
    h = _dot_t(acts, w1[:, NUM_BELIEF:].astype(BF))
    hsig = _dot_t(sig_ref[...], w1[:, :NUM_BELIEF])
    h3 = h.reshape(nb, A, F)
    kmask = lax.broadcasted_iota(jnp.int32, (nb, A, F), 1) == 0
    h3 = (h3 + jnp.where(kmask, hsig[:, None, :], 0.0)).astype(BF)

    attc = wp_ref[_AW:_AW + F, _ATT_LANE:_ATT_LANE + 4]
    b1 = wp_ref[_B1:_B1 + 1, :].astype(BF)
    b2 = wp_ref[_B2:_B2 + 1, :].astype(BF)
    x = _gat_block(h3, attc[:, 0:1], attc[:, 1:2], b1, nb)
    h2 = _dot_t(x.reshape(nb * A, F),
                wp_ref[_W2:_W2 + F, :].astype(BF)).astype(BF).reshape(nb, A, F)
    x2 = _gat_block(h2, attc[:, 2:3], attc[:, 3:4], b2, nb)

    ego = x2[:, 0, :].astype(jnp.float32)
    v = _dot_t(ego, wp_ref[_VW:_VW + F, :]) + wp_ref[_VB:_VB + 1, :]
    f = _dot_t(v, wp_ref[_OW:_OW + F, :]) + wp_ref[_OB:_OB + 1, :]
    z = _dot_t(f, wp_ref[_LW:_LW + 64, :]) + wp_ref[_LB:_LB + 1, :64]
    o = _dot_t(z, wp_ref[_AW:_AW + 2 * F, :64])
    apb = jnp.concatenate([wp_ref[_APB:_APB + 1, :],
                           wp_ref[_APB + 1:_APB + 2, :]], axis=1)
    out_ref[...] = o + apb


@jax.jit
def kernel(signals, neighbor_actions, W1, att_src1, att_dst1, b1, W2,
           att_src2, att_dst2, b2, in_w, in_b, out_w, out_b, lm_w, lm_b,
           ap_w, ap_b):
    B = signals.shape[0]
    NB = 256
    grid = (B // NB,)

    na3 = neighbor_actions.reshape(B, NUM_AGENTS, ACTION_DIM)

    att4 = jnp.stack([att_src1.reshape(-1), att_dst1.reshape(-1),
                      att_src2.reshape(-1), att_dst2.reshape(-1)], axis=1)
    aw_block = jnp.concatenate([
        ap_w,
        jnp.zeros((2 * FEAT, _ATT_LANE - 64), jnp.float32),
        jnp.concatenate([att4, jnp.zeros((FEAT, 4), jnp.float32)], 0),
        jnp.zeros((2 * FEAT, 128 - _ATT_LANE - 4), jnp.float32),
    ], axis=1)

    def row(v):
        r = jnp.zeros((8, FEAT), jnp.float32)
        return r.at[0, :v.shape[0]].set(v)

    apb_rows = jnp.zeros((_ROWS - _APB, FEAT), jnp.float32)
    apb_rows = apb_rows.at[0, :].set(ap_b[:FEAT]).at[1, :].set(ap_b[FEAT:])

    wpack = jnp.concatenate([
        W1, W2, in_w[2 * FEAT:], out_w, lm_w, aw_block,
        row(b1), row(b2), row(in_b[2 * FEAT:]), row(out_b), row(lm_b),
        apb_rows,
    ], axis=0)

    out = pl.pallas_call(
        _tgnn_kernel,
        grid=grid,
        in_specs=[
            pl.BlockSpec((NB, NUM_BELIEF), lambda i: (i, 0)),
            pl.BlockSpec((NB, NUM_AGENTS, ACTION_DIM), lambda i: (i, 0, 0)),
            pl.BlockSpec((_ROWS, FEAT), lambda i: (0, 0)),
        ],
        out_specs=pl.BlockSpec((NB, ACTION_DIM * NUM_AGENTS), lambda i: (i, 0)),
        out_shape=jax.ShapeDtypeStruct((B, ACTION_DIM * NUM_AGENTS), jnp.float32),
    )(signals, na3, wpack)
    return out


# speedup vs baseline: 1.1246x; 1.1246x over previous
import jax
import jax.numpy as jnp
from jax import lax
from jax.experimental import pallas as pl

NUM_AGENTS = 32
ACTION_DIM = 8
NUM_BELIEF = 120
HIDDEN = 32
HEADS = 4
FEAT = HIDDEN * HEADS

BF = jnp.bfloat16

_W1, _W2, _VW, _OW, _LW, _AW = 0, 128, 256, 384, 512, 576
_B1, _B2, _VB, _OB, _LB, _APB = 832, 840, 848, 856, 864, 872
_ROWS = 888
_ATT_LANE = 72

_DN_T = (((1,), (1,)), ((), ()))


def _dot_t(x, w, out_dtype=jnp.float32):
    return lax.dot_general(x, w, _DN_T, preferred_element_type=out_dtype)


def _gat_block(h3, asf, adf, bias, nb):
    A, C, F = NUM_AGENTS, HIDDEN, FEAT
    hflat = h3.reshape(nb * A, F)
    bi = lax.broadcasted_iota(jnp.int32, (F, F), 0) // C
    bj = lax.broadcasted_iota(jnp.int32, (F, F), 1) // C
    BD = (bi == bj).astype(jnp.float32)
    BDb = BD.astype(BF)
    a_src_bc = jnp.dot(hflat, (asf * BD).astype(BF),
                       preferred_element_type=jnp.float32).astype(BF)
    a_dst_bc = jnp.dot(hflat, (adf * BD).astype(BF),
                       preferred_element_type=jnp.float32).astype(BF)
    sel = (lax.broadcasted_iota(jnp.int32, (A, F), 0)
           == lax.broadcasted_iota(jnp.int32, (A, F), 1) % C).astype(BF)
    asrcT = jnp.sum(a_src_bc.reshape(nb, A, F) * sel[None], axis=1)
    L = asrcT[:, None, :] + a_dst_bc.reshape(nb, A, F)
    L = jnp.maximum(L, jnp.asarray(0.2, BF) * L)
    j_io = lax.broadcasted_iota(jnp.int32, (nb, A, F), 1)
    i_io = lax.broadcasted_iota(jnp.int32, (nb, A, F), 2) % C
    E = jnp.where(j_io == i_io, jnp.asarray(0.0, BF), jnp.exp(L))
    den = jnp.dot(E.reshape(nb * A, F), BDb,
                  preferred_element_type=jnp.float32).reshape(nb, A, F)
    rden = jnp.ones((), BF) / den.astype(BF)
    Hbig = jnp.concatenate([h3, h3, h3, h3], axis=1)
    ri = lax.broadcasted_iota(jnp.int32, (nb, F, F), 1) // C
    ci = lax.broadcasted_iota(jnp.int32, (nb, F, F), 2) // C
    Hbig = jnp.where(ri == ci, Hbig, jnp.asarray(0.0, BF))
    outU = lax.dot_general(E, Hbig, (((2,), (1,)), ((0,), (0,))),
                           preferred_element_type=jnp.float32)
    out = outU.astype(BF) * rden + bias[None, :, :]
    return jnp.maximum(out, jnp.asarray(0.0, BF))


def _tgnn_kernel(sig_ref, na_ref, wp_ref, out_ref):
    nb = sig_ref.shape[0]
    A, F = NUM_AGENTS, FEAT
    w1 = wp_ref[_W1:_W1 + F, :]
    acts = na_ref[...].reshape(nb * A, ACTION_DIM).astype(BF)
    h = _dot_t(acts, w1[:, NUM_BELIEF:].astype(BF))
    hsig = _dot_t(sig_ref[...], w1[:, :NUM_BELIEF])
    h3 = h.reshape(nb, A, F)
    kmask = lax.broadcasted_iota(jnp.int32, (nb, A, F), 1) == 0
    h3 = (h3 + jnp.where(kmask, hsig[:, None, :], 0.0)).astype(BF)

    attc = wp_ref[_AW:_AW + F, _ATT_LANE:_ATT_LANE + 4]
    b1 = wp_ref[_B1:_B1 + 1, :].astype(BF)
    b2 = wp_ref[_B2:_B2 + 1, :].astype(BF)
    x = _gat_block(h3, attc[:, 0:1], attc[:, 1:2], b1, nb)
    h2 = _dot_t(x.reshape(nb * A, F),
                wp_ref[_W2:_W2 + F, :].astype(BF)).astype(BF).reshape(nb, A, F)
    x2 = _gat_block(h2, attc[:, 2:3], attc[:, 3:4], b2, nb)

    ego = x2[:, 0, :].astype(jnp.float32)
    v = _dot_t(ego, wp_ref[_VW:_VW + F, :]) + wp_ref[_VB:_VB + 1, :]
    f = _dot_t(v, wp_ref[_OW:_OW + F, :]) + wp_ref[_OB:_OB + 1, :]
    z = _dot_t(f, wp_ref[_LW:_LW + 64, :]) + wp_ref[_LB:_LB + 1, :64]
    o = _dot_t(z, wp_ref[_AW:_AW + 2 * F, :64])
    apb = jnp.concatenate([wp_ref[_APB:_APB + 1, :],
                           wp_ref[_APB + 1:_APB + 2, :]], axis=1)
    out_ref[...] = o + apb


@jax.jit
def kernel(signals, neighbor_actions, W1, att_src1, att_dst1, b1, W2,
           att_src2, att_dst2, b2, in_w, in_b, out_w, out_b, lm_w, lm_b,
           ap_w, ap_b):
    B = signals.shape[0]
    NB = 256
    grid = (B // NB,)


    def place(piece, r0, c0=0):
        rr, cc = piece.shape
        return jnp.pad(piece, ((r0, _ROWS - r0 - rr), (c0, FEAT - c0 - cc)))

    att4 = jnp.stack([att_src1.reshape(-1), att_dst1.reshape(-1),
                      att_src2.reshape(-1), att_dst2.reshape(-1)], axis=1)
    wpack = (place(W1, _W1) + place(W2, _W2) + place(in_w[2 * FEAT:], _VW)
             + place(out_w, _OW) + place(lm_w, _LW) + place(ap_w, _AW)
             + place(att4, _AW, _ATT_LANE)
             + place(b1.reshape(1, -1), _B1) + place(b2.reshape(1, -1), _B2)
             + place(in_b[2 * FEAT:].reshape(1, -1), _VB)
             + place(out_b.reshape(1, -1), _OB)
             + place(lm_b.reshape(1, -1), _LB)
             + place(ap_b.reshape(2, FEAT), _APB))

    out = pl.pallas_call(
        _tgnn_kernel,
        grid=grid,
        in_specs=[
            pl.BlockSpec((NB, NUM_BELIEF), lambda i: (i, 0)),
            pl.BlockSpec((NB, NUM_AGENTS, ACTION_DIM), lambda i: (i, 0, 0)),
            pl.BlockSpec((_ROWS, FEAT), lambda i: (0, 0)),
        ],
        out_specs=pl.BlockSpec((NB, ACTION_DIM * NUM_AGENTS), lambda i: (i, 0)),
        out_shape=jax.ShapeDtypeStruct((B, ACTION_DIM * NUM_AGENTS), jnp.float32),
    )(signals, neighbor_actions.reshape(B, NUM_AGENTS, ACTION_DIM), wpack)
    return out


# speedup vs baseline: 1.3164x; 1.1705x over previous
import jax
import jax.numpy as jnp
from jax import lax
from jax.experimental import pallas as pl

NUM_AGENTS = 32
ACTION_DIM = 8
NUM_BELIEF = 120
HIDDEN = 32
HEADS = 4
FEAT = HIDDEN * HEADS

BF = jnp.bfloat16

_W1, _W2, _VW, _OW, _LW, _AW = 0, 128, 256, 384, 512, 576
_B1, _B2, _VB, _OB, _LB, _APB = 832, 840, 848, 856, 864, 872
_ROWS = 888
_ATT_LANE = 72

_DN_T = (((1,), (1,)), ((), ()))


def _dot_t(x, w, out_dtype=jnp.float32):
    return lax.dot_general(x, w, _DN_T, preferred_element_type=out_dtype)


def _gat_block(h3, asf, adf, bias, nb):
    A, C, F = NUM_AGENTS, HIDDEN, FEAT
    hflat = h3.reshape(nb * A, F)
    bi = lax.broadcasted_iota(jnp.int32, (F, F), 0) // C
    bj = lax.broadcasted_iota(jnp.int32, (F, F), 1) // C
    BD = (bi == bj).astype(jnp.float32)
    BDb = BD.astype(BF)
    a_src_bc = jnp.dot(hflat, (asf * BD).astype(BF),
                       preferred_element_type=jnp.float32).astype(BF)
    a_dst_bc = jnp.dot(hflat, (adf * BD).astype(BF),
                       preferred_element_type=jnp.float32).astype(BF)
    sel = (lax.broadcasted_iota(jnp.int32, (A, F), 0)
           == lax.broadcasted_iota(jnp.int32, (A, F), 1) % C).astype(BF)
    asrcT = jnp.sum(a_src_bc.reshape(nb, A, F) * sel[None], axis=1)
    L = asrcT[:, None, :] + a_dst_bc.reshape(nb, A, F)
    L = jnp.maximum(L, jnp.asarray(0.2, BF) * L)
    j_io = lax.broadcasted_iota(jnp.int32, (nb, A, F), 1)
    i_io = lax.broadcasted_iota(jnp.int32, (nb, A, F), 2) % C
    E = jnp.where(j_io == i_io, jnp.asarray(0.0, BF), jnp.exp(L))
    den = jnp.dot(E.reshape(nb * A, F), BDb,
                  preferred_element_type=jnp.float32).reshape(nb, A, F)
    rden = jnp.ones((), BF) / den.astype(BF)
    Hbig = jnp.concatenate([h3, h3, h3, h3], axis=1)
    ri = lax.broadcasted_iota(jnp.int32, (nb, F, F), 1) // C
    ci = lax.broadcasted_iota(jnp.int32, (nb, F, F), 2) // C
    Hbig = jnp.where(ri == ci, Hbig, jnp.asarray(0.0, BF))
    outU = lax.dot_general(E, Hbig, (((2,), (1,)), ((0,), (0,))),
                           preferred_element_type=jnp.float32)
    out = outU.astype(BF) * rden + bias[None, :, :]
    return jnp.maximum(out, jnp.asarray(0.0, BF))


def _tgnn_kernel(sig_ref, na_ref, wp_ref, out_ref):
    nb = sig_ref.shape[0]
    A, F = NUM_AGENTS, FEAT
    w1 = wp_ref[_W1:_W1 + F, :]
    na_bf = na_ref[...].astype(BF)
    w1a = w1[:, NUM_BELIEF:].astype(BF)
    hks = [_dot_t(na_bf[:, ACTION_DIM * k:ACTION_DIM * (k + 1)], w1a)[:, None, :]
           for k in range(A)]
    hact = jnp.concatenate(hks, axis=1)
    hsig = _dot_t(sig_ref[...], w1[:, :NUM_BELIEF])
    kmask = lax.broadcasted_iota(jnp.int32, (nb, A, F), 1) == 0
    h3 = (hact + jnp.where(kmask, hsig[:, None, :], 0.0)).astype(BF)

    attc = wp_ref[_AW:_AW + F, _ATT_LANE:_ATT_LANE + 4]
    b1 = wp_ref[_B1:_B1 + 1, :].astype(BF)
    b2 = wp_ref[_B2:_B2 + 1, :].astype(BF)
    x = _gat_block(h3, attc[:, 0:1], attc[:, 1:2], b1, nb)
    h2 = _dot_t(x.reshape(nb * A, F),
                wp_ref[_W2:_W2 + F, :].astype(BF)).astype(BF).reshape(nb, A, F)
    x2 = _gat_block(h2, attc[:, 2:3], attc[:, 3:4], b2, nb)

    ego = x2[:, 0, :].astype(jnp.float32)
    v = _dot_t(ego, wp_ref[_VW:_VW + F, :]) + wp_ref[_VB:_VB + 1, :]
    f = _dot_t(v, wp_ref[_OW:_OW + F, :]) + wp_ref[_OB:_OB + 1, :]
    z = _dot_t(f, wp_ref[_LW:_LW + 64, :]) + wp_ref[_LB:_LB + 1, :64]
    o = _dot_t(z, wp_ref[_AW:_AW + 2 * F, :64])
    apb = jnp.concatenate([wp_ref[_APB:_APB + 1, :],
                           wp_ref[_APB + 1:_APB + 2, :]], axis=1)
    out_ref[...] = o + apb


@jax.jit
def kernel(signals, neighbor_actions, W1, att_src1, att_dst1, b1, W2,
           att_src2, att_dst2, b2, in_w, in_b, out_w, out_b, lm_w, lm_b,
           ap_w, ap_b):
    B = signals.shape[0]
    NB = 256
    grid = (B // NB,)


    def place(piece, r0, c0=0):
        rr, cc = piece.shape
        return jnp.pad(piece, ((r0, _ROWS - r0 - rr), (c0, FEAT - c0 - cc)))

    att4 = jnp.stack([att_src1.reshape(-1), att_dst1.reshape(-1),
                      att_src2.reshape(-1), att_dst2.reshape(-1)], axis=1)
    wpack = (place(W1, _W1) + place(W2, _W2) + place(in_w[2 * FEAT:], _VW)
             + place(out_w, _OW) + place(lm_w, _LW) + place(ap_w, _AW)
             + place(att4, _AW, _ATT_LANE)
             + place(b1.reshape(1, -1), _B1) + place(b2.reshape(1, -1), _B2)
             + place(in_b[2 * FEAT:].reshape(1, -1), _VB)
             + place(out_b.reshape(1, -1), _OB)
             + place(lm_b.reshape(1, -1), _LB)
             + place(ap_b.reshape(2, FEAT), _APB))

    out = pl.pallas_call(
        _tgnn_kernel,
        grid=grid,
        in_specs=[
            pl.BlockSpec((NB, NUM_BELIEF), lambda i: (i, 0)),
            pl.BlockSpec((NB, NUM_AGENTS * ACTION_DIM), lambda i: (i, 0)),
            pl.BlockSpec((_ROWS, FEAT), lambda i: (0, 0)),
        ],
        out_specs=pl.BlockSpec((NB, ACTION_DIM * NUM_AGENTS), lambda i: (i, 0)),
        out_shape=jax.ShapeDtypeStruct((B, ACTION_DIM * NUM_AGENTS), jnp.float32),
    )(signals, neighbor_actions, wpack)
    return out
